# baseline (speedup 1.0000x reference)
import functools

import jax
import jax.numpy as jnp
from jax import lax
from jax.experimental import pallas as pl
from jax.experimental.pallas import tpu as pltpu
from jax.experimental.pallas import tpu_sc as plsc

_EMBED = 64
_HIDDEN = 128
_BATCH = 16384
_PAIR = 2 * _EMBED

_NC = 2
_NS = 16
_NW = _NC * _NS
_BPW = _BATCH // _NW
_CH = 128
_NCH = _BPW // _CH


def _sc_gather_pairs(table_p, idx):
  mesh = plsc.VectorSubcoreMesh(core_axis_name="c", subcore_axis_name="s")

  @functools.partial(
      pl.kernel,
      mesh=mesh,
      out_type=jax.ShapeDtypeStruct((_BATCH, _PAIR), jnp.float32),
      scratch_types=[
          pltpu.VMEM((_NCH, _CH), jnp.int32),
          pltpu.VMEM((_BPW, _PAIR), jnp.float32),
          pltpu.SemaphoreType.DMA,
      ],
  )
  def k(table_hbm, idx_hbm, out_hbm, idx_v, rows_v, sem):
    wid = lax.axis_index("s") * _NC + lax.axis_index("c")
    base = wid * _BPW
    pltpu.sync_copy(idx_hbm.at[wid], idx_v)
    cps = [
        pltpu.async_copy(
            table_hbm.at[idx_v.at[j]],
            rows_v.at[pl.ds(j * _CH, _CH)],
            sem,
        )
        for j in range(_NCH)
    ]
    for cp in cps:
      cp.wait()
    pltpu.sync_copy(rows_v, out_hbm.at[pl.ds(base, _BPW)])

  return k(table_p, idx)


def _mlp_body(xp_ref, s_ref, w1_ref, b1_ref, w2_ref, b2_ref, out_ref):
  xa = xp_ref[:, :_EMBED]
  xb = xp_ref[:, _EMBED:]
  s = s_ref[...]
  ha = jnp.dot(xa, w1_ref[...], preferred_element_type=jnp.float32)
  hb = jnp.dot(xb, w1_ref[...], preferred_element_type=jnp.float32)
  h = jnp.maximum(ha + (hb - ha) * s + b1_ref[...], 0.0)
  out = jnp.dot(h, w2_ref[...], preferred_element_type=jnp.float32)
  out_ref[...] = out + b2_ref[...]


def _tc_mlp(xp, s_col, W1, b1, W2, b2):
  bb = 2048
  return pl.pallas_call(
      _mlp_body,
      grid=(_BATCH // bb,),
      in_specs=[
          pl.BlockSpec((bb, _PAIR), lambda i: (i, 0)),
          pl.BlockSpec((bb, 1), lambda i: (i, 0)),
          pl.BlockSpec((_EMBED, _HIDDEN), lambda i: (0, 0)),
          pl.BlockSpec((1, _HIDDEN), lambda i: (0, 0)),
          pl.BlockSpec((_HIDDEN, _EMBED), lambda i: (0, 0)),
          pl.BlockSpec((1, _EMBED), lambda i: (0, 0)),
      ],
      out_specs=pl.BlockSpec((bb, _EMBED), lambda i: (i, 0)),
      out_shape=jax.ShapeDtypeStruct((_BATCH, _EMBED), jnp.float32),
  )(xp, s_col, W1, b1.reshape(1, _HIDDEN), W2, b2.reshape(1, _EMBED))


def kernel(nodes, table, W1, b1, W2, b2):
  nodes_i = nodes.astype(jnp.int32)
  table_p = table.reshape(table.shape[0] // 2, _PAIR)
  idx = (nodes_i >> 1).reshape(_NW, _NCH, _CH)
  s_col = (nodes_i & 1).astype(jnp.float32).reshape(_BATCH, 1)
  xp = _sc_gather_pairs(table_p, idx)
  return _tc_mlp(xp, s_col, W1, b1, W2, b2)


# speedup vs baseline: 2.4549x; 2.4549x over previous
import functools

import jax
import jax.numpy as jnp
from jax import lax
from jax.experimental import pallas as pl
from jax.experimental.pallas import tpu as pltpu
from jax.experimental.pallas import tpu_sc as plsc

_EMBED = 64
_HIDDEN = 128
_BATCH = 16384

_NC = 2
_NS = 16
_NW = _NC * _NS
_BPW = _BATCH // _NW


def _sc_gather(table3, idx):
  mesh = plsc.VectorSubcoreMesh(core_axis_name="c", subcore_axis_name="s")

  @functools.partial(
      pl.kernel,
      mesh=mesh,
      out_type=jax.ShapeDtypeStruct((_BATCH // 8, 8, _EMBED), jnp.float32),
      scratch_types=[
          pltpu.VMEM((_BPW,), jnp.int32),
          pltpu.VMEM((_BPW // 8, 8, _EMBED), jnp.float32),
          pltpu.SemaphoreType.DMA,
      ],
  )
  def k(table_hbm, idx_hbm, out_hbm, idx_v, rows_v, sem):
    wid = lax.axis_index("s") * _NC + lax.axis_index("c")
    pltpu.sync_copy(idx_hbm.at[wid], idx_v)

    def body(g, carry):
      v = idx_v[pl.ds(g * 16, 16)]
      for l in range(16):
        t = v[l]
        pltpu.make_async_copy(
            table_hbm.at[t >> 3, t & 7],
            rows_v.at[2 * g + l // 8, l % 8],
            sem,
        ).start()

      @pl.when(g >= 2)
      def _():
        pltpu.make_async_copy(
            table_hbm.at[pl.ds(0, 2)],
            rows_v.at[pl.ds((g - 2) * 2, 2)],
            sem,
        ).wait()

      return carry

    ngroups = _BPW // 16
    lax.fori_loop(0, ngroups, body, 0)
    pltpu.make_async_copy(
        table_hbm.at[pl.ds(0, 4)],
        rows_v.at[pl.ds(_BPW // 8 - 4, 4)],
        sem,
    ).wait()
    pltpu.sync_copy(rows_v, out_hbm.at[pl.ds(wid * (_BPW // 8), _BPW // 8)])

  return k(table3, idx)


def _mlp_body(x_ref, w1_ref, b1_ref, w2_ref, b2_ref, out_ref):
  x = x_ref[...]
  h = jnp.dot(x, w1_ref[...], preferred_element_type=jnp.float32)
  h = jnp.maximum(h + b1_ref[...], 0.0)
  out = jnp.dot(h, w2_ref[...], preferred_element_type=jnp.float32)
  out_ref[...] = out + b2_ref[...]


def _tc_mlp(x, W1, b1, W2, b2):
  bb = 2048
  return pl.pallas_call(
      _mlp_body,
      grid=(_BATCH // bb,),
      in_specs=[
          pl.BlockSpec((bb, _EMBED), lambda i: (i, 0)),
          pl.BlockSpec((_EMBED, _HIDDEN), lambda i: (0, 0)),
          pl.BlockSpec((1, _HIDDEN), lambda i: (0, 0)),
          pl.BlockSpec((_HIDDEN, _EMBED), lambda i: (0, 0)),
          pl.BlockSpec((1, _EMBED), lambda i: (0, 0)),
      ],
      out_specs=pl.BlockSpec((bb, _EMBED), lambda i: (i, 0)),
      out_shape=jax.ShapeDtypeStruct((_BATCH, _EMBED), jnp.float32),
  )(x, W1, b1.reshape(1, _HIDDEN), W2, b2.reshape(1, _EMBED))


def kernel(nodes, table, W1, b1, W2, b2):
  idx = nodes.astype(jnp.int32).reshape(_NW, _BPW)
  table3 = table.reshape(table.shape[0] // 8, 8, _EMBED)
  x = _sc_gather(table3, idx).reshape(_BATCH, _EMBED)
  return _tc_mlp(x, W1, b1, W2, b2)


# speedup vs baseline: 2.5045x; 1.0202x over previous
import functools

import jax
import jax.numpy as jnp
from jax import lax
from jax.experimental import pallas as pl
from jax.experimental.pallas import tpu as pltpu
from jax.experimental.pallas import tpu_sc as plsc

_EMBED = 64
_HIDDEN = 128
_BATCH = 16384

_NC = 2
_NS = 16
_NW = _NC * _NS
_BPW = _BATCH // _NW


def _sc_gather(table3, idx):
  mesh = plsc.VectorSubcoreMesh(core_axis_name="c", subcore_axis_name="s")

  @functools.partial(
      pl.kernel,
      mesh=mesh,
      out_type=jax.ShapeDtypeStruct((_BATCH // 8, 8, _EMBED), jnp.float32),
      scratch_types=[
          pltpu.VMEM((_BPW,), jnp.int32),
          pltpu.VMEM((_BPW // 8, 8, _EMBED), jnp.float32),
          pltpu.SemaphoreType.DMA,
      ],
  )
  def k(table_hbm, idx_hbm, out_hbm, idx_v, rows_v, sem):
    wid = lax.axis_index("s") * _NC + lax.axis_index("c")
    pltpu.sync_copy(idx_hbm.at[wid], idx_v)

    def body(g, carry):
      v = idx_v[pl.ds(g * 16, 16)]
      for l in range(16):
        t = v[l]
        pltpu.make_async_copy(
            table_hbm.at[t >> 3, t & 7],
            rows_v.at[2 * g + l // 8, l % 8],
            sem,
        ).start()

      @pl.when(g >= 4)
      def _():
        pltpu.make_async_copy(
            table_hbm.at[pl.ds(0, 2)],
            rows_v.at[pl.ds((g - 4) * 2, 2)],
            sem,
        ).wait()

      return carry

    ngroups = _BPW // 16
    lax.fori_loop(0, ngroups, body, 0)
    pltpu.make_async_copy(
        table_hbm.at[pl.ds(0, 8)],
        rows_v.at[pl.ds(_BPW // 8 - 8, 8)],
        sem,
    ).wait()
    pltpu.sync_copy(rows_v, out_hbm.at[pl.ds(wid * (_BPW // 8), _BPW // 8)])

  return k(table3, idx)


def _mlp_body(x_ref, w1_ref, b1_ref, w2_ref, b2_ref, out_ref):
  x = x_ref[...]
  h = jnp.dot(x, w1_ref[...], preferred_element_type=jnp.float32)
  h = jnp.maximum(h + b1_ref[...], 0.0)
  out = jnp.dot(h, w2_ref[...], preferred_element_type=jnp.float32)
  out_ref[...] = out + b2_ref[...]


def _tc_mlp(x, W1, b1, W2, b2):
  bb = 4096
  return pl.pallas_call(
      _mlp_body,
      grid=(_BATCH // bb,),
      in_specs=[
          pl.BlockSpec((bb, _EMBED), lambda i: (i, 0)),
          pl.BlockSpec((_EMBED, _HIDDEN), lambda i: (0, 0)),
          pl.BlockSpec((1, _HIDDEN), lambda i: (0, 0)),
          pl.BlockSpec((_HIDDEN, _EMBED), lambda i: (0, 0)),
          pl.BlockSpec((1, _EMBED), lambda i: (0, 0)),
      ],
      out_specs=pl.BlockSpec((bb, _EMBED), lambda i: (i, 0)),
      out_shape=jax.ShapeDtypeStruct((_BATCH, _EMBED), jnp.float32),
  )(x, W1, b1.reshape(1, _HIDDEN), W2, b2.reshape(1, _EMBED))


def kernel(nodes, table, W1, b1, W2, b2):
  idx = nodes.astype(jnp.int32).reshape(_NW, _BPW)
  table3 = table.reshape(table.shape[0] // 8, 8, _EMBED)
  x = _sc_gather(table3, idx).reshape(_BATCH, _EMBED)
  return _tc_mlp(x, W1, b1, W2, b2)


# speedup vs baseline: 2.5208x; 1.0065x over previous
import functools

import jax
import jax.numpy as jnp
from jax import lax
from jax.experimental import pallas as pl
from jax.experimental.pallas import tpu as pltpu
from jax.experimental.pallas import tpu_sc as plsc

_EMBED = 64
_HIDDEN = 128
_BATCH = 16384

_NC = 2
_NS = 16
_NW = _NC * _NS
_BPW = _BATCH // _NW


def _sc_gather(table3, idx):
  mesh = plsc.VectorSubcoreMesh(core_axis_name="c", subcore_axis_name="s")

  @functools.partial(
      pl.kernel,
      mesh=mesh,
      out_type=jax.ShapeDtypeStruct((_BATCH // 8, 8, _EMBED), jnp.float32),
      scratch_types=[
          pltpu.VMEM((_BPW,), jnp.int32),
          pltpu.VMEM((_BPW // 8, 8, _EMBED), jnp.float32),
          pltpu.SemaphoreType.DMA,
      ],
  )
  def k(table_hbm, idx_hbm, out_hbm, idx_v, rows_v, sem):
    wid = lax.axis_index("s") * _NC + lax.axis_index("c")
    pltpu.sync_copy(idx_hbm.at[wid], idx_v)

    def body(g, carry):
      for h in range(2):
        v = idx_v[pl.ds(g * 32 + h * 16, 16)]
        for l in range(16):
          t = v[l]
          pltpu.make_async_copy(
              table_hbm.at[t >> 3, t & 7],
              rows_v.at[4 * g + 2 * h + l // 8, l % 8],
              sem,
          ).start()

      @pl.when(g >= 2)
      def _():
        pltpu.make_async_copy(
            table_hbm.at[pl.ds(0, 4)],
            rows_v.at[pl.ds((g - 2) * 4, 4)],
            sem,
        ).wait()

      return carry

    ngroups = _BPW // 32
    lax.fori_loop(0, ngroups, body, 0)
    pltpu.make_async_copy(
        table_hbm.at[pl.ds(0, 8)],
        rows_v.at[pl.ds(_BPW // 8 - 8, 8)],
        sem,
    ).wait()
    pltpu.sync_copy(rows_v, out_hbm.at[pl.ds(wid * (_BPW // 8), _BPW // 8)])

  return k(table3, idx)


def _mlp_body(x_ref, w1_ref, b1_ref, w2_ref, b2_ref, out_ref):
  x = x_ref[...]
  h = jnp.dot(x, w1_ref[...], preferred_element_type=jnp.float32)
  h = jnp.maximum(h + b1_ref[...], 0.0)
  out = jnp.dot(h, w2_ref[...], preferred_element_type=jnp.float32)
  out_ref[...] = out + b2_ref[...]


def _tc_mlp(x, W1, b1, W2, b2):
  bb = 8192
  return pl.pallas_call(
      _mlp_body,
      grid=(_BATCH // bb,),
      in_specs=[
          pl.BlockSpec((bb, _EMBED), lambda i: (i, 0)),
          pl.BlockSpec((_EMBED, _HIDDEN), lambda i: (0, 0)),
          pl.BlockSpec((1, _HIDDEN), lambda i: (0, 0)),
          pl.BlockSpec((_HIDDEN, _EMBED), lambda i: (0, 0)),
          pl.BlockSpec((1, _EMBED), lambda i: (0, 0)),
      ],
      out_specs=pl.BlockSpec((bb, _EMBED), lambda i: (i, 0)),
      out_shape=jax.ShapeDtypeStruct((_BATCH, _EMBED), jnp.float32),
  )(x, W1, b1.reshape(1, _HIDDEN), W2, b2.reshape(1, _EMBED))


def kernel(nodes, table, W1, b1, W2, b2):
  idx = nodes.astype(jnp.int32).reshape(_NW, _BPW)
  table3 = table.reshape(table.shape[0] // 8, 8, _EMBED)
  x = _sc_gather(table3, idx).reshape(_BATCH, _EMBED)
  return _tc_mlp(x, W1, b1, W2, b2)


# speedup vs baseline: 2.6163x; 1.0379x over previous
import functools

import jax
import jax.numpy as jnp
from jax import lax
from jax.experimental import pallas as pl
from jax.experimental.pallas import tpu as pltpu
from jax.experimental.pallas import tpu_sc as plsc

_EMBED = 64
_HIDDEN = 128
_BATCH = 16384

_NC = 2
_NS = 16
_NW = _NC * _NS
_BPW = _BATCH // _NW


def _sc_gather(table3, idx):
  mesh = plsc.VectorSubcoreMesh(core_axis_name="c", subcore_axis_name="s")

  @functools.partial(
      pl.kernel,
      mesh=mesh,
      out_type=jax.ShapeDtypeStruct((_BATCH // 8, 8, _EMBED), jnp.float32),
      scratch_types=[
          pltpu.VMEM((_BPW,), jnp.int32),
          pltpu.VMEM((_BPW // 8, 8, _EMBED), jnp.float32),
          pltpu.SemaphoreType.DMA,
      ],
  )
  def k(table_hbm, idx_hbm, out_hbm, idx_v, rows_v, sem):
    wid = lax.axis_index("s") * _NC + lax.axis_index("c")
    pltpu.sync_copy(idx_hbm.at[wid], idx_v)

    def body(g, carry):
      for h in range(2):
        v = idx_v[pl.ds(g * 32 + h * 16, 16)]
        for l in range(16):
          t = v[l]
          pltpu.make_async_copy(
              table_hbm.at[t >> 3, t & 7],
              rows_v.at[4 * g + 2 * h + l // 8, l % 8],
              sem,
          ).start()

      @pl.when(g >= 2)
      def _():
        pltpu.make_async_copy(
            table_hbm.at[pl.ds(0, 4)],
            rows_v.at[pl.ds((g - 2) * 4, 4)],
            sem,
        ).wait()

      return carry

    ngroups = _BPW // 32
    lax.fori_loop(0, ngroups, body, 0)
    pltpu.make_async_copy(
        table_hbm.at[pl.ds(0, 8)],
        rows_v.at[pl.ds(_BPW // 8 - 8, 8)],
        sem,
    ).wait()
    pltpu.sync_copy(rows_v, out_hbm.at[pl.ds(wid * (_BPW // 8), _BPW // 8)])

  return k(table3, idx)


def _mlp_body(x_ref, w1t_ref, b1_ref, w2t_ref, b2_ref, out_ref):
  h = lax.dot_general(
      w1t_ref[...], x_ref[...], (((1,), (1,)), ((), ())),
      preferred_element_type=jnp.float32,
  )
  h = jnp.maximum(h + b1_ref[...], 0.0)
  out = jnp.dot(w2t_ref[...], h, preferred_element_type=jnp.float32)
  out_ref[...] = out + b2_ref[...]


def _tc_mlp_t(x, W1t, b1, W2t, b2):
  bb = 8192
  return pl.pallas_call(
      _mlp_body,
      grid=(_BATCH // bb,),
      in_specs=[
          pl.BlockSpec((bb, _EMBED), lambda i: (i, 0)),
          pl.BlockSpec((_HIDDEN, _EMBED), lambda i: (0, 0)),
          pl.BlockSpec((_HIDDEN, 1), lambda i: (0, 0)),
          pl.BlockSpec((_EMBED, _HIDDEN), lambda i: (0, 0)),
          pl.BlockSpec((_EMBED, 1), lambda i: (0, 0)),
      ],
      out_specs=pl.BlockSpec((_EMBED, bb), lambda i: (0, i)),
      out_shape=jax.ShapeDtypeStruct((_EMBED, _BATCH), jnp.float32),
  )(x, W1t, b1.reshape(_HIDDEN, 1), W2t, b2.reshape(_EMBED, 1))


def kernel(nodes, table, W1, b1, W2, b2):
  idx = nodes.astype(jnp.int32).reshape(_NW, _BPW)
  table3 = table.reshape(table.shape[0] // 8, 8, _EMBED)
  x = _sc_gather(table3, idx).reshape(_BATCH, _EMBED)
  return _tc_mlp_t(x, W1.T, b1, W2.T, b2).T
